# baseline (speedup 1.0000x reference)
import functools

import jax
import jax.numpy as jnp
from jax import lax
from jax.experimental import pallas as pl
from jax.experimental.pallas import tpu as pltpu
from jax.experimental.pallas import tpu_sc as plsc

N = 10000
E = 320000
D = 128

N_PAD = 10240
NW = 32
CHUNK = 80
NBUF = 4
N_CHUNKS = 128
E_CHUNKS = E // CHUNK
NQ = 8
SHCH = N_CHUNKS // NQ
NTQ = 2
ROWS_PER_TILE = N_PAD // 16
ZB = CHUNK


def _tc1_body(x_ref, g_ref, b_ref, wl_ref, wr_ref, bl_ref, y_ref, z_ref):
    x = x_ref[...]
    mean = jnp.mean(x, axis=1, keepdims=True)
    xc = x - mean
    var = jnp.mean(xc * xc, axis=1, keepdims=True)
    xn = xc * lax.rsqrt(var + 1e-5) * g_ref[...] + b_ref[...]
    y_ref[...] = jnp.dot(xn, wl_ref[...], preferred_element_type=jnp.float32)
    z_ref[...] = (
        jnp.dot(xn, wr_ref[...], preferred_element_type=jnp.float32) + bl_ref[...]
    )


def _tc1(x, gamma, beta, W_l, b_l, W_r):
    blk = 1000
    grid = N // blk
    return pl.pallas_call(
        _tc1_body,
        grid=(grid,),
        in_specs=[
            pl.BlockSpec((blk, D), lambda i: (i, 0)),
            pl.BlockSpec((1, D), lambda i: (0, 0)),
            pl.BlockSpec((1, D), lambda i: (0, 0)),
            pl.BlockSpec((D, D), lambda i: (0, 0)),
            pl.BlockSpec((D, D), lambda i: (0, 0)),
            pl.BlockSpec((1, D), lambda i: (0, 0)),
        ],
        out_specs=[
            pl.BlockSpec((blk, D), lambda i: (i, 0)),
            pl.BlockSpec((blk, D), lambda i: (i, 0)),
        ],
        out_shape=[
            jax.ShapeDtypeStruct((N, D), jnp.float32),
            jax.ShapeDtypeStruct((N, D), jnp.float32),
        ],
    )(x, gamma.reshape(1, D), beta.reshape(1, D), W_l, W_r, b_l.reshape(1, D))


def _sc_body(y_hbm, e_hbm, tsrc_hbm, tdst_hbm, ones_hbm, zdeg_hbm,
             acc_out, deg_out,
             src_v, dst_v, rows_v, ones_v,
             acc_sh, deg_sh, gsems, ssems, dsems):
    c = lax.axis_index("c")
    s = lax.axis_index("s")
    w = c * 16 + s

    pltpu.sync_copy(ones_hbm, ones_v)

    def zfill(i, _):

        for j in range(D // 16):
            rows_v[0, i, pl.ds(j * 16, 16)] = jnp.zeros((16,), jnp.float32)
        return 0
    lax.fori_loop(0, ZB, zfill, 0)

    row0 = s * ROWS_PER_TILE

    def zcopy(k, _):
        pltpu.sync_copy(rows_v.at[0], acc_sh.at[pl.ds(row0 + k * ZB, ZB)])
        return 0
    lax.fori_loop(0, ROWS_PER_TILE // ZB, zcopy, 0)
    pltpu.sync_copy(zdeg_hbm, deg_sh.at[pl.ds(row0, ROWS_PER_TILE)])

    plsc.subcore_barrier()

    def wait_scatters():
        for b in range(NBUF):
            pltpu.make_async_copy(
                rows_v.at[b], acc_sh.at[dst_v.at[0]], ssems.at[b]).wait()
            pltpu.make_async_copy(
                ones_v, deg_sh.at[dst_v.at[0]], dsems.at[b]).wait()

    def group(g, _):
        k0 = g * NBUF

        @pl.when(g > 0)
        def _():
            wait_scatters()

        gd = []
        for b in range(NBUF):
            gd.append(pltpu.async_copy(
                y_hbm.at[src_v.at[k0 + b]], rows_v.at[b], gsems.at[b]))
        for b in range(NBUF):
            gd[b].wait()
            pltpu.async_copy(
                rows_v.at[b], acc_sh.at[dst_v.at[k0 + b]], ssems.at[b],
                add=True)
            pltpu.async_copy(
                ones_v, deg_sh.at[dst_v.at[k0 + b]], dsems.at[b], add=True)
        return 0

    for h in range(NQ):
        q0 = w * N_CHUNKS + h * SHCH
        if h < NTQ:
            pltpu.sync_copy(e_hbm.at[0, pl.ds(q0, SHCH)], src_v)
            pltpu.sync_copy(e_hbm.at[1, pl.ds(q0, SHCH)], dst_v)
        else:
            @pl.when(w < NW - 1)
            def _():
                pltpu.sync_copy(e_hbm.at[0, pl.ds(q0, SHCH)], src_v)
                pltpu.sync_copy(e_hbm.at[1, pl.ds(q0, SHCH)], dst_v)

            @pl.when(w == NW - 1)
            def _():
                pltpu.sync_copy(tsrc_hbm.at[h - NTQ], src_v)
                pltpu.sync_copy(tdst_hbm.at[h - NTQ], dst_v)
        lax.fori_loop(0, SHCH // NBUF, group, 0)
        wait_scatters()

    plsc.subcore_barrier()

    def wcopy(k, _):
        r = row0 + k * ZB
        pltpu.sync_copy(acc_sh.at[pl.ds(r, ZB)], acc_out.at[c, pl.ds(r, ZB)])
        return 0
    lax.fori_loop(0, ROWS_PER_TILE // ZB, wcopy, 0)
    pltpu.sync_copy(deg_sh.at[pl.ds(row0, ROWS_PER_TILE)],
                    deg_out.at[c, pl.ds(row0, ROWS_PER_TILE)])


def _sc_aggregate(y, edges, tsrc, tdst, ones_c, zdeg_c):
    mesh = plsc.VectorSubcoreMesh(core_axis_name="c", subcore_axis_name="s")
    kern = pl.kernel(
        _sc_body,
        out_type=[
            jax.ShapeDtypeStruct((2, N_PAD, D), jnp.float32),
            jax.ShapeDtypeStruct((2, N_PAD), jnp.float32),
        ],
        mesh=mesh,
        scratch_types=[
            pltpu.VMEM((SHCH, CHUNK), jnp.int32),
            pltpu.VMEM((SHCH, CHUNK), jnp.int32),
            pltpu.VMEM((NBUF, CHUNK, D), jnp.float32),
            pltpu.VMEM((CHUNK,), jnp.float32),
            pltpu.VMEM_SHARED((N_PAD, D), jnp.float32),
            pltpu.VMEM_SHARED((N_PAD,), jnp.float32),
            pltpu.SemaphoreType.DMA((NBUF,)),
            pltpu.SemaphoreType.DMA((NBUF,)),
            pltpu.SemaphoreType.DMA((NBUF,)),
        ],
    )
    return kern(y, edges, tsrc, tdst, ones_c, zdeg_c)


def _tc2_body(acc_ref, deg_ref, z_ref, out_ref):
    a = acc_ref[0] + acc_ref[1]
    d = deg_ref[0] + deg_ref[1]
    d = jnp.maximum(d, 1.0)
    out_ref[...] = jnp.maximum(a / d + z_ref[...], 0.0)


def _tc2(acc, deg, z):
    blk = 1000
    grid = N // blk
    return pl.pallas_call(
        _tc2_body,
        grid=(grid,),
        in_specs=[
            pl.BlockSpec((2, blk, D), lambda i: (0, i, 0)),
            pl.BlockSpec((2, blk, 1), lambda i: (0, i, 0)),
            pl.BlockSpec((blk, D), lambda i: (i, 0)),
        ],
        out_specs=pl.BlockSpec((blk, D), lambda i: (i, 0)),
        out_shape=jax.ShapeDtypeStruct((N, D), jnp.float32),
    )(acc, deg.reshape(2, N_PAD, 1), z)


@jax.jit
def kernel(node_feature, edge_index, gamma, beta, W_l, b_l, W_r):
    y, z = _tc1(node_feature, gamma, beta, W_l, b_l, W_r)

    npad = NQ - NTQ
    i = jnp.arange(npad * SHCH * CHUNK, dtype=jnp.int32)
    tsrc = (i % N).reshape(npad, SHCH, CHUNK)
    tdst = (N + i % (N_PAD - N)).reshape(npad, SHCH, CHUNK)
    edges = edge_index.reshape(2, E_CHUNKS, CHUNK)
    ones_c = jnp.ones((CHUNK,), jnp.float32)
    zdeg_c = jnp.zeros((ROWS_PER_TILE,), jnp.float32)

    acc, deg = _sc_aggregate(y, edges, tsrc, tdst, ones_c, zdeg_c)
    out = _tc2(acc, deg, z)
    return out


# speedup vs baseline: 1.0137x; 1.0137x over previous
import functools

import jax
import jax.numpy as jnp
from jax import lax
from jax.experimental import pallas as pl
from jax.experimental.pallas import tpu as pltpu
from jax.experimental.pallas import tpu_sc as plsc

N = 10000
E = 320000
D = 128

N_PAD = 10240
NW = 32
CHUNK = 80
NBUF = 4
N_CHUNKS = 128
E_CHUNKS = E // CHUNK
NQ = 8
SHCH = N_CHUNKS // NQ
NTQ = 2
ROWS_PER_TILE = N_PAD // 16
ZB = CHUNK


def _tc1_body(x_ref, g_ref, b_ref, wl_ref, wr_ref, bl_ref, y_ref, z_ref):
    x = x_ref[...]
    mean = jnp.mean(x, axis=1, keepdims=True)
    xc = x - mean
    var = jnp.mean(xc * xc, axis=1, keepdims=True)
    xn = xc * lax.rsqrt(var + 1e-5) * g_ref[...] + b_ref[...]
    y_ref[...] = jnp.dot(xn, wl_ref[...], preferred_element_type=jnp.float32)
    z_ref[...] = (
        jnp.dot(xn, wr_ref[...], preferred_element_type=jnp.float32) + bl_ref[...]
    )


def _tc1(x, gamma, beta, W_l, b_l, W_r):
    blk = 1000
    grid = N // blk
    return pl.pallas_call(
        _tc1_body,
        grid=(grid,),
        in_specs=[
            pl.BlockSpec((blk, D), lambda i: (i, 0)),
            pl.BlockSpec((1, D), lambda i: (0, 0)),
            pl.BlockSpec((1, D), lambda i: (0, 0)),
            pl.BlockSpec((D, D), lambda i: (0, 0)),
            pl.BlockSpec((D, D), lambda i: (0, 0)),
            pl.BlockSpec((1, D), lambda i: (0, 0)),
        ],
        out_specs=[
            pl.BlockSpec((blk, D), lambda i: (i, 0)),
            pl.BlockSpec((blk, D), lambda i: (i, 0)),
        ],
        out_shape=[
            jax.ShapeDtypeStruct((N, D), jnp.float32),
            jax.ShapeDtypeStruct((N, D), jnp.float32),
        ],
    )(x, gamma.reshape(1, D), beta.reshape(1, D), W_l, W_r, b_l.reshape(1, D))


def _sc_body(y_hbm, e_hbm, tsrc_hbm, tdst_hbm, acc_out, deg_out,
             src_v, dst_v, rows_v, ones_v, zdeg_v,
             acc_sh, deg_sh, gsems, ssems, dsems):
    c = lax.axis_index("c")
    s = lax.axis_index("s")
    w = c * 16 + s

    def fill16(i, _):
        ones_v[pl.ds(i * 16, 16)] = jnp.ones((16,), jnp.float32)
        return 0
    lax.fori_loop(0, CHUNK // 16, fill16, 0)

    def zfill(i, _):
        for j in range(D // 16):
            rows_v[0, i, pl.ds(j * 16, 16)] = jnp.zeros((16,), jnp.float32)
        return 0
    lax.fori_loop(0, ZB, zfill, 0)

    def zdeg_fill(i, _):
        zdeg_v[pl.ds(i * 16, 16)] = jnp.zeros((16,), jnp.float32)
        return 0
    lax.fori_loop(0, ROWS_PER_TILE // 16, zdeg_fill, 0)

    row0 = s * ROWS_PER_TILE

    def zcopy(k, _):
        pltpu.sync_copy(rows_v.at[0], acc_sh.at[pl.ds(row0 + k * ZB, ZB)])
        return 0
    lax.fori_loop(0, ROWS_PER_TILE // ZB, zcopy, 0)
    pltpu.sync_copy(zdeg_v, deg_sh.at[pl.ds(row0, ROWS_PER_TILE)])

    plsc.subcore_barrier()

    def group(g, _):
        k0 = g * NBUF
        gd = []
        for b in range(NBUF):
            gd.append(pltpu.async_copy(
                y_hbm.at[src_v.at[k0 + b]], rows_v.at[b], gsems.at[b]))
        sd = []
        for b in range(NBUF):
            gd[b].wait()
            sd.append(pltpu.async_copy(
                rows_v.at[b], acc_sh.at[dst_v.at[k0 + b]], ssems.at[b],
                add=True))
            sd.append(pltpu.async_copy(
                ones_v, deg_sh.at[dst_v.at[k0 + b]], dsems.at[b], add=True))
        for d in sd:
            d.wait()
        return 0

    for h in range(NQ):
        q0 = w * N_CHUNKS + h * SHCH
        if h < NTQ:
            pltpu.sync_copy(e_hbm.at[0, pl.ds(q0, SHCH)], src_v)
            pltpu.sync_copy(e_hbm.at[1, pl.ds(q0, SHCH)], dst_v)
        else:
            @pl.when(w < NW - 1)
            def _():
                pltpu.sync_copy(e_hbm.at[0, pl.ds(q0, SHCH)], src_v)
                pltpu.sync_copy(e_hbm.at[1, pl.ds(q0, SHCH)], dst_v)

            @pl.when(w == NW - 1)
            def _():
                pltpu.sync_copy(tsrc_hbm.at[h - NTQ], src_v)
                pltpu.sync_copy(tdst_hbm.at[h - NTQ], dst_v)
        lax.fori_loop(0, SHCH // NBUF, group, 0)

    plsc.subcore_barrier()

    def wcopy(k, _):
        r = row0 + k * ZB
        pltpu.sync_copy(acc_sh.at[pl.ds(r, ZB)], acc_out.at[c, pl.ds(r, ZB)])
        return 0
    lax.fori_loop(0, ROWS_PER_TILE // ZB, wcopy, 0)
    pltpu.sync_copy(deg_sh.at[pl.ds(row0, ROWS_PER_TILE)],
                    deg_out.at[c, pl.ds(row0, ROWS_PER_TILE)])


def _sc_aggregate(y, edges, tsrc, tdst):
    mesh = plsc.VectorSubcoreMesh(core_axis_name="c", subcore_axis_name="s")
    kern = pl.kernel(
        _sc_body,
        out_type=[
            jax.ShapeDtypeStruct((2, N_PAD, D), jnp.float32),
            jax.ShapeDtypeStruct((2, N_PAD), jnp.float32),
        ],
        mesh=mesh,
        scratch_types=[
            pltpu.VMEM((SHCH, CHUNK), jnp.int32),
            pltpu.VMEM((SHCH, CHUNK), jnp.int32),
            pltpu.VMEM((NBUF, CHUNK, D), jnp.float32),
            pltpu.VMEM((CHUNK,), jnp.float32),
            pltpu.VMEM((ROWS_PER_TILE,), jnp.float32),
            pltpu.VMEM_SHARED((N_PAD, D), jnp.float32),
            pltpu.VMEM_SHARED((N_PAD,), jnp.float32),
            pltpu.SemaphoreType.DMA((NBUF,)),
            pltpu.SemaphoreType.DMA((NBUF,)),
            pltpu.SemaphoreType.DMA((NBUF,)),
        ],
    )
    return kern(y, edges, tsrc, tdst)


def _tc2_body(acc_ref, deg_ref, z_ref, out_ref):
    a = acc_ref[0] + acc_ref[1]
    d = deg_ref[0] + deg_ref[1]
    d = jnp.maximum(d, 1.0)
    out_ref[...] = jnp.maximum(a / d + z_ref[...], 0.0)


def _tc2(acc, deg, z):
    blk = 1000
    grid = N // blk
    return pl.pallas_call(
        _tc2_body,
        grid=(grid,),
        in_specs=[
            pl.BlockSpec((2, blk, D), lambda i: (0, i, 0)),
            pl.BlockSpec((2, blk, 1), lambda i: (0, i, 0)),
            pl.BlockSpec((blk, D), lambda i: (i, 0)),
        ],
        out_specs=pl.BlockSpec((blk, D), lambda i: (i, 0)),
        out_shape=jax.ShapeDtypeStruct((N, D), jnp.float32),
    )(acc, deg.reshape(2, N_PAD, 1), z)


@jax.jit
def kernel(node_feature, edge_index, gamma, beta, W_l, b_l, W_r):
    y, z = _tc1(node_feature, gamma, beta, W_l, b_l, W_r)

    npad = NQ - NTQ
    i = jnp.arange(npad * SHCH * CHUNK, dtype=jnp.int32)
    tsrc = (i % N).reshape(npad, SHCH, CHUNK)
    tdst = (N + i % (N_PAD - N)).reshape(npad, SHCH, CHUNK)
    edges = edge_index.reshape(2, E_CHUNKS, CHUNK)

    acc, deg = _sc_aggregate(y, edges, tsrc, tdst)
    out = _tc2(acc, deg, z)
    return out
